# baseline (speedup 1.0000x reference)
import functools

import jax
import jax.numpy as jnp
from jax import lax
from jax.experimental import pallas as pl
from jax.experimental.pallas import tpu as pltpu
from jax.experimental.pallas import tpu_sc as plsc

_T = 8192
_H = 2048
_E = 8
_K = 2
_BT = 512
_BM = 256
_BN = 512
_PADT = _T * _K + _E * _BM
_MTOT = _PADT // _BM
_NC = 2
_NS = 16
_L = 16
_TPW = _T // _NS
_NCH = _TPW // _L
_TRASH = _PADT
_TRASH_T = _T
_SPAN = 1152
_GPW = _PADT // (_NC * _NS)
_GCH = 24
_CPW = _T // (_NC * _NS)
_CCH = 16


def _router_kernel(x_ref, gate_ref, logits_ref, e1_ref, e2_ref, w1_ref, w2_ref):
    x = x_ref[...]
    logits = lax.dot_general(x, gate_ref[...], (((1,), (1,)), ((), ())),
                             preferred_element_type=jnp.float32)
    logits_ref[...] = logits
    ids = lax.broadcasted_iota(jnp.int32, logits.shape, 1)
    m1 = jnp.max(logits, axis=1, keepdims=True)
    e1 = jnp.min(jnp.where(logits == m1, ids, _E), axis=1, keepdims=True)
    mask1 = ids == e1
    l2 = jnp.where(mask1, -jnp.inf, logits)
    m2 = jnp.max(l2, axis=1, keepdims=True)
    e2 = jnp.min(jnp.where(l2 == m2, ids, _E), axis=1, keepdims=True)
    r = jnp.exp(m2 - m1)
    w1 = 1.0 / (1.0 + r)
    e1_ref[...] = e1
    e2_ref[...] = e2
    w1_ref[...] = w1
    w2_ref[...] = 1.0 - w1


def _router(x, gate_w):
    return pl.pallas_call(
        _router_kernel,
        grid=(_T // _BT,),
        in_specs=[
            pl.BlockSpec((_BT, _H), lambda i: (i, 0)),
            pl.BlockSpec((_E, _H), lambda i: (0, 0)),
        ],
        out_specs=[
            pl.BlockSpec((_BT, _E), lambda i: (i, 0)),
            pl.BlockSpec((_BT, 1), lambda i: (i, 0)),
            pl.BlockSpec((_BT, 1), lambda i: (i, 0)),
            pl.BlockSpec((_BT, 1), lambda i: (i, 0)),
            pl.BlockSpec((_BT, 1), lambda i: (i, 0)),
        ],
        out_shape=[
            jax.ShapeDtypeStruct((_T, _E), jnp.float32),
            jax.ShapeDtypeStruct((_T, 1), jnp.int32),
            jax.ShapeDtypeStruct((_T, 1), jnp.int32),
            jax.ShapeDtypeStruct((_T, 1), jnp.float32),
            jax.ShapeDtypeStruct((_T, 1), jnp.float32),
        ],
    )(x, gate_w)


def _vtake(x, idx):
    return lax.gather(
        x, idx[:, None],
        lax.GatherDimensionNumbers(offset_dims=(), collapsed_slice_dims=(0,),
                                   start_index_map=(0,)),
        (1,), mode=lax.GatherScatterMode.PROMISE_IN_BOUNDS)


def _vcumsum(x, iota, zeros16):
    for sh in (1, 2, 4, 8):
        g = _vtake(x, jnp.maximum(iota - sh, zeros16))
        x = x + jnp.where(iota >= sh, g, zeros16)
    return x


def _vsum_splat(x, iota, zeros16):
    return _vtake(_vcumsum(x, iota, zeros16), jnp.full((_L,), _L - 1))


def _dispatch_body(e1_hbm, e2_hbm, w1_hbm, w2_hbm,
                   srctok_hbm, wsrt_hbm, d1_hbm, d2_hbm, eid_hbm, cnts_hbm,
                   e1v, e2v, w1v, w2v,
                   destv, tokv, wvalv, d1iv, d2iv,
                   lanev, basev, offsv, totv, paddv,
                   initidx, zeroi, zerof, eidv):
    c = lax.axis_index("c")
    s = lax.axis_index("s")
    tok0 = s * _TPW
    iota = lax.iota(jnp.int32, _L)

    pltpu.sync_copy(e1_hbm.at[pl.ds(tok0, _TPW)], e1v)
    pltpu.sync_copy(e2_hbm.at[pl.ds(tok0, _TPW)], e2v)
    pltpu.sync_copy(w1_hbm.at[pl.ds(tok0, _TPW)], w1v)
    pltpu.sync_copy(w2_hbm.at[pl.ds(tok0, _TPW)], w2v)

    ones16 = jnp.full((_L,), 1, jnp.int32)
    zeros16 = jnp.zeros((_L,), jnp.int32)
    counts = zeros16
    for k in range(_NCH):
        for ev_ref in (e1v, e2v):
            ev = ev_ref[pl.ds(k * _L, _L)]
            for e in range(_E):
                cnt = _vsum_splat(jnp.where(ev == e, ones16, zeros16), iota, zeros16)
                counts = counts + jnp.where(iota == e, cnt, zeros16)

    lanev[...] = counts
    pltpu.sync_copy(lanev, cnts_hbm.at[c * _NS + s])
    plsc.subcore_barrier()
    tot = jnp.zeros((_L,), jnp.int32)
    base = jnp.zeros((_L,), jnp.int32)
    for s2 in range(_NS):
        pltpu.sync_copy(cnts_hbm.at[c * _NS + s2], lanev)
        row = lanev[...]
        tot = tot + row
        sel = jnp.full((_L,), jnp.where(s2 < s, 1, 0))
        base = base + row * sel
    padded = (tot + (_BM - 1)) & (-_BM)
    incl = _vcumsum(padded, iota, zeros16)
    offs = incl - padded
    totv[...] = tot
    paddv[...] = padded
    offsv[...] = offs
    mybase = offs + base

    c_splat = jnp.full((_L,), c.astype(jnp.int32))
    offs4 = _vtake(offs, jnp.full((_L,), 4, jnp.int32))
    regstart = offs4 * c_splat
    regend = offs4 + (jnp.full((_L,), _PADT, jnp.int32) - offs4) * c_splat

    for jj in range(_L // 2):
        zeroi[pl.ds(jj * _L, _L)] = jnp.zeros((_L,), jnp.int32)
        zerof[pl.ds(jj * _L, _L)] = jnp.zeros((_L,), jnp.float32)
    for r in range(_SPAN // 128):
        for jj in range(8):
            pos = regstart + jnp.full((_L,), s * _SPAN + (r * 8 + jj) * _L) + iota
            pos = jnp.where(pos < regend, pos, jnp.full((_L,), _TRASH))
            initidx[r, pl.ds(jj * _L, _L)] = pos
    for r in range(_SPAN // 128):
        pltpu.sync_copy(zeroi, srctok_hbm.at[initidx.at[r]])
        pltpu.sync_copy(zerof, wsrt_hbm.at[initidx.at[r]])
    plsc.subcore_barrier()

    @pl.when((c == 0) & (s == 4))
    def _eid():
        for ch in range(5):
            bs = (jnp.full((_L,), ch * _L) + iota) * _BM
            eid = jnp.full((_L,), -1, jnp.int32)
            for e in range(_E):
                lo = _vtake(offs, jnp.full((_L,), e, jnp.int32))
                hi = lo + _vtake(padded, jnp.full((_L,), e, jnp.int32))
                ev_ = jnp.full((_L,), e, jnp.int32)
                eid = jnp.where(bs >= lo, jnp.where(bs < hi, ev_, eid), eid)
            eidv[pl.ds(ch * _L, _L)] = eid
        pltpu.sync_copy(eidv, eid_hbm)

    basecur = mybase
    for arr_i, (ev_ref, wv_ref) in enumerate(((e1v, w1v), (e2v, w2v))):
        for k in range(_NCH):
            ev = ev_ref[pl.ds(k * _L, _L)]
            bases = _vtake(basecur, ev)
            rank = zeros16
            for e in range(_E):
                m = ev == e
                mi = jnp.where(m, ones16, zeros16)
                cs = _vcumsum(mi, iota, zeros16)
                rank = jnp.where(m, cs - 1, rank)
                cnt = _vtake(cs, jnp.full((_L,), _L - 1))
                basecur = basecur + jnp.where(iota == e, cnt, zeros16)
            dest = bases + rank
            owned = lax.shift_right_logical(ev, 2) == c_splat
            destf = jnp.where(owned, dest, jnp.full((_L,), _TRASH))
            tid = jnp.full((_L,), tok0 + k * _L) + iota
            i = arr_i * _TPW + k * _L
            destv[i // 128, pl.ds(i % 128, _L)] = destf
            tokv[i // 128, pl.ds(i % 128, _L)] = tid
            wvalv[i // 128, pl.ds(i % 128, _L)] = wv_ref[pl.ds(k * _L, _L)]
            div = d1iv if arr_i == 0 else d2iv
            j = k * _L
            div[j // 128, pl.ds(j % 128, _L)] = jnp.where(
                owned, tid, jnp.full((_L,), _TRASH_T))
    for r in range(8):
        pltpu.sync_copy(tokv.at[r], srctok_hbm.at[destv.at[r]])
        pltpu.sync_copy(wvalv.at[r], wsrt_hbm.at[destv.at[r]])
    for r in range(4):
        pltpu.sync_copy(destv.at[r], d1_hbm.at[d1iv.at[r]])
        pltpu.sync_copy(destv.at[4 + r], d2_hbm.at[d2iv.at[r]])


def _dispatch(e1r, e2r, w1r, w2r):
    mesh = plsc.VectorSubcoreMesh(core_axis_name="c", subcore_axis_name="s")
    f = pl.kernel(
        _dispatch_body,
        out_type=(
            jax.ShapeDtypeStruct((_PADT + 16,), jnp.int32),
            jax.ShapeDtypeStruct((_PADT + 16,), jnp.float32),
            jax.ShapeDtypeStruct((_T + 16,), jnp.int32),
            jax.ShapeDtypeStruct((_T + 16,), jnp.int32),
            jax.ShapeDtypeStruct((80,), jnp.int32),
            jax.ShapeDtypeStruct((_NC * _NS, _L), jnp.int32),
        ),
        mesh=mesh,
        scratch_types=[
            pltpu.VMEM((_TPW,), jnp.int32),
            pltpu.VMEM((_TPW,), jnp.int32),
            pltpu.VMEM((_TPW,), jnp.float32),
            pltpu.VMEM((_TPW,), jnp.float32),
            pltpu.VMEM((8, 128), jnp.int32),
            pltpu.VMEM((8, 128), jnp.int32),
            pltpu.VMEM((8, 128), jnp.float32),
            pltpu.VMEM((4, 128), jnp.int32),
            pltpu.VMEM((4, 128), jnp.int32),
            pltpu.VMEM((_L,), jnp.int32),
            pltpu.VMEM((_L,), jnp.int32),
            pltpu.VMEM((_L,), jnp.int32),
            pltpu.VMEM((_L,), jnp.int32),
            pltpu.VMEM((_L,), jnp.int32),
            pltpu.VMEM((_SPAN // 128, 128), jnp.int32),
            pltpu.VMEM((128,), jnp.int32),
            pltpu.VMEM((128,), jnp.float32),
            pltpu.VMEM((80,), jnp.int32),
        ],
    )
    return f(e1r, e2r, w1r, w2r)


def _gather_body(src_hbm, x_hbm, xs_hbm, idxv, rowsA, rowsB, semA, semB):
    c = lax.axis_index("c")
    s = lax.axis_index("s")
    w = s * _NC + c
    base = w * _GPW
    pltpu.sync_copy(src_hbm.at[pl.ds(base, _GPW)], idxv)
    nch = _GPW // _GCH
    bufs = (rowsA, rowsB)
    sems = (semA, semB)
    copies = []
    for t in range(nch):
        cp = pltpu.make_async_copy(
            x_hbm.at[idxv.at[pl.ds(t * _GCH, _GCH)]], bufs[t % 2], sems[t % 2])
        cp.start()
        copies.append(cp)
        if t >= 1:
            copies[t - 1].wait()
            pltpu.sync_copy(bufs[(t - 1) % 2],
                            xs_hbm.at[pl.ds(base + (t - 1) * _GCH, _GCH)])
    copies[nch - 1].wait()
    pltpu.sync_copy(bufs[(nch - 1) % 2],
                    xs_hbm.at[pl.ds(base + (nch - 1) * _GCH, _GCH)])


def _gather(srctok, x):
    mesh = plsc.VectorSubcoreMesh(core_axis_name="c", subcore_axis_name="s")
    f = pl.kernel(
        _gather_body,
        out_type=jax.ShapeDtypeStruct((_PADT, _H), jnp.float32),
        mesh=mesh,
        scratch_types=[
            pltpu.VMEM((_GPW,), jnp.int32),
            pltpu.VMEM((_GCH, _H), jnp.float32),
            pltpu.VMEM((_GCH, _H), jnp.float32),
            pltpu.SemaphoreType.DMA,
            pltpu.SemaphoreType.DMA,
        ],
    )
    return f(srctok, x)


def _gmm_kernel(eid_ref, xs_ref, w_ref, wsrt_ref, ys_ref):
    m = pl.program_id(1)

    @pl.when(eid_ref[m] >= 0)
    def _():
        y = lax.dot_general(xs_ref[...], w_ref[0], (((1,), (1,)), ((), ())),
                            preferred_element_type=jnp.float32)
        ys_ref[...] = wsrt_ref[...] * y


def _gmm(eidm, xs, expert_w, wsrt2):
    grid_spec = pltpu.PrefetchScalarGridSpec(
        num_scalar_prefetch=1,
        grid=(_H // _BN, _MTOT),
        in_specs=[
            pl.BlockSpec((_BM, _H), lambda n, m, eid: (m, 0)),
            pl.BlockSpec((1, _BN, _H),
                         lambda n, m, eid: (jnp.maximum(eid[m], 0), n, 0)),
            pl.BlockSpec((_BM, 1), lambda n, m, eid: (m, 0)),
        ],
        out_specs=pl.BlockSpec((_BM, _BN), lambda n, m, eid: (m, n)),
    )
    return pl.pallas_call(
        _gmm_kernel,
        grid_spec=grid_spec,
        out_shape=jax.ShapeDtypeStruct((_PADT, _H), jnp.float32),
        compiler_params=pltpu.CompilerParams(
            dimension_semantics=("arbitrary", "arbitrary"),
        ),
    )(eidm, xs, expert_w, wsrt2)


def _cgather_body(ys_hbm, d1_hbm, d2_hbm, g1_hbm, g2_hbm,
                  i1v, i2v, rowsA, rowsB, semA, semB):
    c = lax.axis_index("c")
    s = lax.axis_index("s")
    w = s * _NC + c
    tok0 = w * _CPW
    pltpu.sync_copy(d1_hbm.at[pl.ds(tok0, _CPW)], i1v)
    pltpu.sync_copy(d2_hbm.at[pl.ds(tok0, _CPW)], i2v)
    bufs = (rowsA, rowsB)
    sems = (semA, semB)
    nch = _CPW // _CCH
    for iv, g_hbm in ((i1v, g1_hbm), (i2v, g2_hbm)):
        copies = []
        for t in range(nch):
            cp = pltpu.make_async_copy(
                ys_hbm.at[iv.at[pl.ds(t * _CCH, _CCH)]], bufs[t % 2], sems[t % 2])
            cp.start()
            copies.append(cp)
            if t >= 1:
                copies[t - 1].wait()
                pltpu.sync_copy(bufs[(t - 1) % 2],
                                g_hbm.at[pl.ds(tok0 + (t - 1) * _CCH, _CCH)])
        copies[nch - 1].wait()
        pltpu.sync_copy(bufs[(nch - 1) % 2],
                        g_hbm.at[pl.ds(tok0 + (nch - 1) * _CCH, _CCH)])


def _combine_gather(ys, d1, d2):
    mesh = plsc.VectorSubcoreMesh(core_axis_name="c", subcore_axis_name="s")
    f = pl.kernel(
        _cgather_body,
        out_type=(
            jax.ShapeDtypeStruct((_T, _H), jnp.float32),
            jax.ShapeDtypeStruct((_T, _H), jnp.float32),
        ),
        mesh=mesh,
        scratch_types=[
            pltpu.VMEM((_CPW,), jnp.int32),
            pltpu.VMEM((_CPW,), jnp.int32),
            pltpu.VMEM((_CCH, _H), jnp.float32),
            pltpu.VMEM((_CCH, _H), jnp.float32),
            pltpu.SemaphoreType.DMA,
            pltpu.SemaphoreType.DMA,
        ],
    )
    return f(ys, d1, d2)


def _add_kernel(a_ref, b_ref, o_ref):
    o_ref[...] = a_ref[...] + b_ref[...]


def _add(a, b):
    return pl.pallas_call(
        _add_kernel,
        grid=(_T // _BT,),
        in_specs=[pl.BlockSpec((_BT, _H), lambda i: (i, 0)),
                  pl.BlockSpec((_BT, _H), lambda i: (i, 0))],
        out_specs=pl.BlockSpec((_BT, _H), lambda i: (i, 0)),
        out_shape=jax.ShapeDtypeStruct((_T, _H), jnp.float32),
    )(a, b)


def kernel(hidden_states, gate_w, expert_w):
    logits, e1, e2, w1, w2 = _router(hidden_states, gate_w)
    srctok, wsrt, d1, d2, eidm, _unused_cnts = _dispatch(
        e1.reshape(_T), e2.reshape(_T), w1.reshape(_T), w2.reshape(_T))
    xs = _gather(srctok, hidden_states)
    wsrt2 = wsrt[:_PADT].reshape(_PADT, 1)
    ys = _gmm(eidm, xs, expert_w, wsrt2)
    g1, g2 = _combine_gather(ys, d1[:_T], d2[:_T])
    out = _add(g1, g2)
    return out, logits


# speedup vs baseline: 1.9381x; 1.9381x over previous
import functools

import jax
import jax.numpy as jnp
from jax import lax
from jax.experimental import pallas as pl
from jax.experimental.pallas import tpu as pltpu
from jax.experimental.pallas import tpu_sc as plsc

_T = 8192
_H = 2048
_E = 8
_K = 2
_BT = 512
_BM = 256
_BN = 512
_PADT = _T * _K + _E * _BM
_MTOT = _PADT // _BM
_NC = 2
_NS = 16
_L = 16
_TPW = _T // _NS
_NCH = _TPW // _L
_TRASH = _PADT
_TRASH_T = _T
_SPAN = 1152
_GPW = _PADT // (_NC * _NS)
_GCH = 24
_CPW = _T // (_NC * _NS)
_CCH = 16


def _router_kernel(x_ref, gate_ref, logits_ref, e1_ref, e2_ref, w1_ref, w2_ref):
    x = x_ref[...]
    logits = lax.dot_general(x, gate_ref[...], (((1,), (1,)), ((), ())),
                             preferred_element_type=jnp.float32)
    logits_ref[...] = logits
    ids = lax.broadcasted_iota(jnp.int32, logits.shape, 1)
    m1 = jnp.max(logits, axis=1, keepdims=True)
    e1 = jnp.min(jnp.where(logits == m1, ids, _E), axis=1, keepdims=True)
    mask1 = ids == e1
    l2 = jnp.where(mask1, -jnp.inf, logits)
    m2 = jnp.max(l2, axis=1, keepdims=True)
    e2 = jnp.min(jnp.where(l2 == m2, ids, _E), axis=1, keepdims=True)
    r = jnp.exp(m2 - m1)
    w1 = 1.0 / (1.0 + r)
    e1_ref[...] = e1
    e2_ref[...] = e2
    w1_ref[...] = w1
    w2_ref[...] = 1.0 - w1


def _router(x, gate_w):
    return pl.pallas_call(
        _router_kernel,
        grid=(_T // _BT,),
        in_specs=[
            pl.BlockSpec((_BT, _H), lambda i: (i, 0)),
            pl.BlockSpec((_E, _H), lambda i: (0, 0)),
        ],
        out_specs=[
            pl.BlockSpec((_BT, _E), lambda i: (i, 0)),
            pl.BlockSpec((_BT, 1), lambda i: (i, 0)),
            pl.BlockSpec((_BT, 1), lambda i: (i, 0)),
            pl.BlockSpec((_BT, 1), lambda i: (i, 0)),
            pl.BlockSpec((_BT, 1), lambda i: (i, 0)),
        ],
        out_shape=[
            jax.ShapeDtypeStruct((_T, _E), jnp.float32),
            jax.ShapeDtypeStruct((_T, 1), jnp.int32),
            jax.ShapeDtypeStruct((_T, 1), jnp.int32),
            jax.ShapeDtypeStruct((_T, 1), jnp.float32),
            jax.ShapeDtypeStruct((_T, 1), jnp.float32),
        ],
    )(x, gate_w)


def _vtake(x, idx):
    return lax.gather(
        x, idx[:, None],
        lax.GatherDimensionNumbers(offset_dims=(), collapsed_slice_dims=(0,),
                                   start_index_map=(0,)),
        (1,), mode=lax.GatherScatterMode.PROMISE_IN_BOUNDS)


def _vcumsum(x, iota, zeros16):
    for sh in (1, 2, 4, 8):
        g = _vtake(x, jnp.maximum(iota - sh, zeros16))
        x = x + jnp.where(iota >= sh, g, zeros16)
    return x


def _vsum_splat(x, iota, zeros16):
    return _vtake(_vcumsum(x, iota, zeros16), jnp.full((_L,), _L - 1))


def _dispatch_body(e1_hbm, e2_hbm, w1_hbm, w2_hbm,
                   srctok_hbm, wsrt_hbm, d1a_hbm, d1b_hbm, d2a_hbm, d2b_hbm,
                   eid_hbm, cnts_hbm,
                   e1v, e2v, w1v, w2v,
                   destv, tokv, wvalv, d1lv, d2lv,
                   lanev, callv, offsv, totv, paddv, eidv, sctsem):
    c = lax.axis_index("c")
    s = lax.axis_index("s")
    tok0 = s * _TPW
    iota = lax.iota(jnp.int32, _L)

    pltpu.sync_copy(e1_hbm.at[pl.ds(tok0, _TPW)], e1v)
    pltpu.sync_copy(e2_hbm.at[pl.ds(tok0, _TPW)], e2v)
    pltpu.sync_copy(w1_hbm.at[pl.ds(tok0, _TPW)], w1v)
    pltpu.sync_copy(w2_hbm.at[pl.ds(tok0, _TPW)], w2v)

    ones16 = jnp.full((_L,), 1, jnp.int32)
    zeros16 = jnp.zeros((_L,), jnp.int32)
    counts = zeros16
    for k in range(_NCH):
        for ev_ref in (e1v, e2v):
            ev = ev_ref[pl.ds(k * _L, _L)]
            for e in range(_E):
                cnt = _vsum_splat(jnp.where(ev == e, ones16, zeros16), iota, zeros16)
                counts = counts + jnp.where(iota == e, cnt, zeros16)

    lanev[...] = counts
    pltpu.sync_copy(lanev, cnts_hbm.at[c * _NS + s])
    plsc.subcore_barrier()
    pltpu.sync_copy(cnts_hbm.at[pl.ds(c * _NS, _NS)], callv)
    tot = jnp.zeros((_L,), jnp.int32)
    base = jnp.zeros((_L,), jnp.int32)
    for s2 in range(_NS):
        row = callv[s2]
        tot = tot + row
        sel = jnp.full((_L,), jnp.where(s2 < s, 1, 0))
        base = base + row * sel
    padded = (tot + (_BM - 1)) & (-_BM)
    incl = _vcumsum(padded, iota, zeros16)
    offs = incl - padded
    totv[...] = tot
    paddv[...] = padded
    offsv[...] = offs
    mybase = offs + base

    c_splat = jnp.full((_L,), c.astype(jnp.int32))

    @pl.when((c == 0) & (s == 4))
    def _eid():
        for ch in range(5):
            bs = (jnp.full((_L,), ch * _L) + iota) * _BM
            eid = jnp.full((_L,), -1, jnp.int32)
            for e in range(_E):
                lo = _vtake(offs, jnp.full((_L,), e, jnp.int32))
                hi = lo + _vtake(padded, jnp.full((_L,), e, jnp.int32))
                ev_ = jnp.full((_L,), e, jnp.int32)
                eid = jnp.where(bs >= lo, jnp.where(bs < hi, ev_, eid), eid)
            eidv[pl.ds(ch * _L, _L)] = eid
        pltpu.sync_copy(eidv, eid_hbm)

    basecur = mybase
    for arr_i, (ev_ref, wv_ref) in enumerate(((e1v, w1v), (e2v, w2v))):
        for k in range(_NCH):
            ev = ev_ref[pl.ds(k * _L, _L)]
            bases = _vtake(basecur, ev)
            rank = zeros16
            for e in range(_E):
                m = ev == e
                mi = jnp.where(m, ones16, zeros16)
                cs = _vcumsum(mi, iota, zeros16)
                rank = jnp.where(m, cs - 1, rank)
                cnt = _vtake(cs, jnp.full((_L,), _L - 1))
                basecur = basecur + jnp.where(iota == e, cnt, zeros16)
            dest = bases + rank
            owned = lax.shift_right_logical(ev, 2) == c_splat
            destf = jnp.where(owned, dest, jnp.full((_L,), _TRASH))
            tid = jnp.full((_L,), tok0 + k * _L) + iota
            i = arr_i * _TPW + k * _L
            destv[i // 128, pl.ds(i % 128, _L)] = destf
            tokv[i // 128, pl.ds(i % 128, _L)] = tid
            wvalv[i // 128, pl.ds(i % 128, _L)] = wv_ref[pl.ds(k * _L, _L)]
            dlv = d1lv if arr_i == 0 else d2lv
            dlv[pl.ds(k * _L, _L)] = destf
    hs = []
    for r in range(8):
        h1 = pltpu.make_async_copy(tokv.at[r], srctok_hbm.at[destv.at[r]], sctsem)
        h1.start()
        hs.append(h1)
        h2 = pltpu.make_async_copy(wvalv.at[r], wsrt_hbm.at[destv.at[r]], sctsem)
        h2.start()
        hs.append(h2)

    @pl.when(c == 0)
    def _wd0():
        pltpu.sync_copy(d1lv, d1a_hbm.at[pl.ds(tok0, _TPW)])
        pltpu.sync_copy(d2lv, d2a_hbm.at[pl.ds(tok0, _TPW)])

    @pl.when(c == 1)
    def _wd1():
        pltpu.sync_copy(d1lv, d1b_hbm.at[pl.ds(tok0, _TPW)])
        pltpu.sync_copy(d2lv, d2b_hbm.at[pl.ds(tok0, _TPW)])
    for h in hs:
        h.wait()


def _dispatch(e1r, e2r, w1r, w2r):
    mesh = plsc.VectorSubcoreMesh(core_axis_name="c", subcore_axis_name="s")
    f = pl.kernel(
        _dispatch_body,
        out_type=(
            jax.ShapeDtypeStruct((_PADT + 16,), jnp.int32),
            jax.ShapeDtypeStruct((_PADT + 16,), jnp.float32),
            jax.ShapeDtypeStruct((_T,), jnp.int32),
            jax.ShapeDtypeStruct((_T,), jnp.int32),
            jax.ShapeDtypeStruct((_T,), jnp.int32),
            jax.ShapeDtypeStruct((_T,), jnp.int32),
            jax.ShapeDtypeStruct((80,), jnp.int32),
            jax.ShapeDtypeStruct((_NC * _NS, _L), jnp.int32),
        ),
        mesh=mesh,
        scratch_types=[
            pltpu.VMEM((_TPW,), jnp.int32),
            pltpu.VMEM((_TPW,), jnp.int32),
            pltpu.VMEM((_TPW,), jnp.float32),
            pltpu.VMEM((_TPW,), jnp.float32),
            pltpu.VMEM((8, 128), jnp.int32),
            pltpu.VMEM((8, 128), jnp.int32),
            pltpu.VMEM((8, 128), jnp.float32),
            pltpu.VMEM((_TPW,), jnp.int32),
            pltpu.VMEM((_TPW,), jnp.int32),
            pltpu.VMEM((_L,), jnp.int32),
            pltpu.VMEM((_NS, _L), jnp.int32),
            pltpu.VMEM((_L,), jnp.int32),
            pltpu.VMEM((_L,), jnp.int32),
            pltpu.VMEM((_L,), jnp.int32),
            pltpu.VMEM((80,), jnp.int32),
            pltpu.SemaphoreType.DMA,
        ],
    )
    return f(e1r, e2r, w1r, w2r)


def _gather_body(src_hbm, x_hbm, xs_hbm, idxv, rowsA, rowsB, semA, semB):
    c = lax.axis_index("c")
    s = lax.axis_index("s")
    w = s * _NC + c
    base = w * _GPW
    pltpu.sync_copy(src_hbm.at[pl.ds(base, _GPW)], idxv)
    zeros16 = jnp.zeros((_L,), jnp.int32)
    tmax = jnp.full((_L,), _T - 1, jnp.int32)
    for k in range(_GPW // _L):
        v = idxv[pl.ds(k * _L, _L)]
        idxv[pl.ds(k * _L, _L)] = jnp.minimum(jnp.maximum(v, zeros16), tmax)
    nch = _GPW // _GCH
    bufs = (rowsA, rowsB)
    sems = (semA, semB)
    copies = []
    for t in range(nch):
        cp = pltpu.make_async_copy(
            x_hbm.at[idxv.at[pl.ds(t * _GCH, _GCH)]], bufs[t % 2], sems[t % 2])
        cp.start()
        copies.append(cp)
        if t >= 1:
            copies[t - 1].wait()
            pltpu.sync_copy(bufs[(t - 1) % 2],
                            xs_hbm.at[pl.ds(base + (t - 1) * _GCH, _GCH)])
    copies[nch - 1].wait()
    pltpu.sync_copy(bufs[(nch - 1) % 2],
                    xs_hbm.at[pl.ds(base + (nch - 1) * _GCH, _GCH)])


def _gather(srctok, x):
    mesh = plsc.VectorSubcoreMesh(core_axis_name="c", subcore_axis_name="s")
    f = pl.kernel(
        _gather_body,
        out_type=jax.ShapeDtypeStruct((_PADT, _H), jnp.float32),
        mesh=mesh,
        scratch_types=[
            pltpu.VMEM((_GPW,), jnp.int32),
            pltpu.VMEM((_GCH, _H), jnp.float32),
            pltpu.VMEM((_GCH, _H), jnp.float32),
            pltpu.SemaphoreType.DMA,
            pltpu.SemaphoreType.DMA,
        ],
    )
    return f(srctok, x)


def _gmm_kernel(eid_ref, xs_ref, w_ref, wsrt_ref, ys_ref):
    m = pl.program_id(1)

    @pl.when(eid_ref[m] >= 0)
    def _():
        y = lax.dot_general(xs_ref[...], w_ref[0], (((1,), (1,)), ((), ())),
                            preferred_element_type=jnp.float32)
        ys_ref[...] = wsrt_ref[...] * y


def _gmm(eidm, xs, expert_w, wsrt2):
    grid_spec = pltpu.PrefetchScalarGridSpec(
        num_scalar_prefetch=1,
        grid=(_H // _BN, _MTOT),
        in_specs=[
            pl.BlockSpec((_BM, _H), lambda n, m, eid: (m, 0)),
            pl.BlockSpec((1, _BN, _H),
                         lambda n, m, eid: (jnp.maximum(eid[m], 0), n, 0)),
            pl.BlockSpec((_BM, 1), lambda n, m, eid: (m, 0)),
        ],
        out_specs=pl.BlockSpec((_BM, _BN), lambda n, m, eid: (m, n)),
    )
    return pl.pallas_call(
        _gmm_kernel,
        grid_spec=grid_spec,
        out_shape=jax.ShapeDtypeStruct((_PADT, _H), jnp.float32),
        compiler_params=pltpu.CompilerParams(
            dimension_semantics=("arbitrary", "arbitrary"),
        ),
    )(eidm, xs, expert_w, wsrt2)


def _cgather_body(ys_hbm, d1a_hbm, d1b_hbm, d2a_hbm, d2b_hbm, e1_hbm, e2_hbm,
                  g1_hbm, g2_hbm,
                  i1v, i2v, iav, ibv, e1v, rowsA, rowsB, semA, semB):
    c = lax.axis_index("c")
    s = lax.axis_index("s")
    w = s * _NC + c
    tok0 = w * _CPW
    four = jnp.full((_L,), 4, jnp.int32)
    for da, db, ehbm, iv in ((d1a_hbm, d1b_hbm, e1_hbm, i1v),
                             (d2a_hbm, d2b_hbm, e2_hbm, i2v)):
        pltpu.sync_copy(da.at[pl.ds(tok0, _CPW)], iav)
        pltpu.sync_copy(db.at[pl.ds(tok0, _CPW)], ibv)
        pltpu.sync_copy(ehbm.at[pl.ds(tok0, _CPW)], e1v)
        for k in range(_CPW // _L):
            sl = pl.ds(k * _L, _L)
            iv[sl] = jnp.where(e1v[sl] < four, iav[sl], ibv[sl])
    bufs = (rowsA, rowsB)
    sems = (semA, semB)
    nch = _CPW // _CCH
    for iv, g_hbm in ((i1v, g1_hbm), (i2v, g2_hbm)):
        copies = []
        for t in range(nch):
            cp = pltpu.make_async_copy(
                ys_hbm.at[iv.at[pl.ds(t * _CCH, _CCH)]], bufs[t % 2], sems[t % 2])
            cp.start()
            copies.append(cp)
            if t >= 1:
                copies[t - 1].wait()
                pltpu.sync_copy(bufs[(t - 1) % 2],
                                g_hbm.at[pl.ds(tok0 + (t - 1) * _CCH, _CCH)])
        copies[nch - 1].wait()
        pltpu.sync_copy(bufs[(nch - 1) % 2],
                        g_hbm.at[pl.ds(tok0 + (nch - 1) * _CCH, _CCH)])


def _combine_gather(ys, d1a, d1b, d2a, d2b, e1r, e2r):
    mesh = plsc.VectorSubcoreMesh(core_axis_name="c", subcore_axis_name="s")
    f = pl.kernel(
        _cgather_body,
        out_type=(
            jax.ShapeDtypeStruct((_T, _H), jnp.float32),
            jax.ShapeDtypeStruct((_T, _H), jnp.float32),
        ),
        mesh=mesh,
        scratch_types=[
            pltpu.VMEM((_CPW,), jnp.int32),
            pltpu.VMEM((_CPW,), jnp.int32),
            pltpu.VMEM((_CPW,), jnp.int32),
            pltpu.VMEM((_CPW,), jnp.int32),
            pltpu.VMEM((_CPW,), jnp.int32),
            pltpu.VMEM((_CCH, _H), jnp.float32),
            pltpu.VMEM((_CCH, _H), jnp.float32),
            pltpu.SemaphoreType.DMA,
            pltpu.SemaphoreType.DMA,
        ],
    )
    return f(ys, d1a, d1b, d2a, d2b, e1r, e2r)


def _add_kernel(a_ref, b_ref, o_ref):
    o_ref[...] = a_ref[...] + b_ref[...]


def _add(a, b):
    return pl.pallas_call(
        _add_kernel,
        grid=(_T // _BT,),
        in_specs=[pl.BlockSpec((_BT, _H), lambda i: (i, 0)),
                  pl.BlockSpec((_BT, _H), lambda i: (i, 0))],
        out_specs=pl.BlockSpec((_BT, _H), lambda i: (i, 0)),
        out_shape=jax.ShapeDtypeStruct((_T, _H), jnp.float32),
    )(a, b)


def kernel(hidden_states, gate_w, expert_w):
    logits, e1, e2, w1, w2 = _router(hidden_states, gate_w)
    e1r = e1.reshape(_T)
    e2r = e2.reshape(_T)
    srctok, wsrt, d1a, d1b, d2a, d2b, eidm, _cnts = _dispatch(
        e1r, e2r, w1.reshape(_T), w2.reshape(_T))
    xs = _gather(srctok, hidden_states)
    wsrt2 = wsrt[:_PADT].reshape(_PADT, 1)
    ys = _gmm(eidm, xs, expert_w, wsrt2)
    g1, g2 = _combine_gather(ys, d1a, d1b, d2a, d2b, e1r, e2r)
    out = _add(g1, g2)
    return out, logits
